# initial kernel scaffold (unmeasured)
import jax
import jax.numpy as jnp
from jax import lax
from jax.experimental import pallas as pl
from jax.experimental.pallas import tpu as pltpu

N_DEV = 8


def kernel(x, w_mat, scale_x, scale_w):
    m, k = x.shape
    n = w_mat.shape[1]
    m_per = m // N_DEV
    scale = (scale_x * scale_w).astype(jnp.float32).reshape(1, 1)

    def body(x_ref, w_ref, s_ref, out_ref, acc_ref, recv_ref, send_sems, recv_sems):
        my = lax.axis_index("i")
        left = lax.rem(my + N_DEV - 1, N_DEV)
        right = lax.rem(my + 1, N_DEV)

        barrier_sem = pltpu.get_barrier_semaphore()
        for nbr in (left, right):
            pl.semaphore_signal(barrier_sem, inc=1, device_id=(nbr,),
                                device_id_type=pl.DeviceIdType.MESH)
        pl.semaphore_wait(barrier_sem, 2)

        acc_ref[...] = jnp.dot(
            x_ref[...].astype(jnp.bfloat16),
            w_ref[...].astype(jnp.bfloat16),
            preferred_element_type=jnp.float32,
        )

        for s in range(N_DEV - 1):
            c_send = lax.rem(my + (N_DEV - 1 - s), N_DEV)
            c_recv = lax.rem(my + (N_DEV - 2 - s), N_DEV)
            rdma = pltpu.make_async_remote_copy(
                src_ref=acc_ref.at[pl.ds(c_send * m_per, m_per), :],
                dst_ref=recv_ref.at[s],
                send_sem=send_sems.at[s],
                recv_sem=recv_sems.at[s],
                device_id=(right,),
                device_id_type=pl.DeviceIdType.MESH,
            )
            rdma.start()
            rdma.wait()
            acc_ref[pl.ds(c_recv * m_per, m_per), :] = (
                acc_ref[pl.ds(c_recv * m_per, m_per), :] + recv_ref[s]
            )

        y = acc_ref[pl.ds(my * m_per, m_per), :] * s_ref[0, 0]
        out_ref[...] = y * jax.nn.sigmoid(y)

    return pl.pallas_call(
        body,
        out_shape=jax.ShapeDtypeStruct((m_per, n), jnp.float32),
        in_specs=[
            pl.BlockSpec(memory_space=pltpu.VMEM),
            pl.BlockSpec(memory_space=pltpu.VMEM),
            pl.BlockSpec(memory_space=pltpu.SMEM),
        ],
        out_specs=pl.BlockSpec(memory_space=pltpu.VMEM),
        scratch_shapes=[
            pltpu.VMEM((m, n), jnp.float32),
            pltpu.VMEM((N_DEV - 1, m_per, n), jnp.float32),
            pltpu.SemaphoreType.DMA((N_DEV - 1,)),
            pltpu.SemaphoreType.DMA((N_DEV - 1,)),
        ],
        compiler_params=pltpu.CompilerParams(collective_id=0),
    )(x, w_mat, scale)


# baseline (device time: 192695 ns/iter reference)
import jax
import jax.numpy as jnp
from jax import lax
from jax.experimental import pallas as pl
from jax.experimental.pallas import tpu as pltpu

N_DEV = 8


def kernel(x, w_mat, scale_x, scale_w):
    m, k = x.shape
    n = w_mat.shape[1]
    m_per = m // N_DEV
    scale = (scale_x * scale_w).astype(jnp.float32).reshape(1, 1)

    def body(x_ref, w_ref, s_ref, out_ref, send_ref, recv_ref, send_sems, recv_sems):
        my = lax.axis_index("i")
        left = lax.rem(my + N_DEV - 1, N_DEV)
        right = lax.rem(my + 1, N_DEV)

        barrier_sem = pltpu.get_barrier_semaphore()
        for nbr in (left, right):
            pl.semaphore_signal(barrier_sem, inc=1, device_id=(nbr,),
                                device_id_type=pl.DeviceIdType.MESH)
        pl.semaphore_wait(barrier_sem, 2)

        def partial(c):
            return jnp.dot(
                x_ref[pl.ds(c * m_per, m_per), :].astype(jnp.bfloat16),
                w_ref[...].astype(jnp.bfloat16),
                preferred_element_type=jnp.float32,
            )

        c0 = lax.rem(my + N_DEV - 1, N_DEV)
        send_ref[0, :, :] = partial(c0).astype(jnp.bfloat16)

        for s in range(N_DEV - 1):
            c_recv = lax.rem(my + (N_DEV - 2 - s), N_DEV)
            rdma = pltpu.make_async_remote_copy(
                src_ref=send_ref.at[s % 2],
                dst_ref=recv_ref.at[s],
                send_sem=send_sems.at[s],
                recv_sem=recv_sems.at[s],
                device_id=(right,),
                device_id_type=pl.DeviceIdType.MESH,
            )
            rdma.start()
            rdma.wait()
            summed = recv_ref[s, :, :].astype(jnp.float32) + partial(c_recv)
            if s < N_DEV - 2:
                send_ref[(s + 1) % 2, :, :] = summed.astype(jnp.bfloat16)
            else:
                y = summed * s_ref[0, 0]
                out_ref[...] = y * jax.nn.sigmoid(y)

    return pl.pallas_call(
        body,
        out_shape=jax.ShapeDtypeStruct((m_per, n), jnp.float32),
        in_specs=[
            pl.BlockSpec(memory_space=pltpu.VMEM),
            pl.BlockSpec(memory_space=pltpu.VMEM),
            pl.BlockSpec(memory_space=pltpu.SMEM),
        ],
        out_specs=pl.BlockSpec(memory_space=pltpu.VMEM),
        scratch_shapes=[
            pltpu.VMEM((2, m_per, n), jnp.bfloat16),
            pltpu.VMEM((N_DEV - 1, m_per, n), jnp.bfloat16),
            pltpu.SemaphoreType.DMA((N_DEV - 1,)),
            pltpu.SemaphoreType.DMA((N_DEV - 1,)),
        ],
        compiler_params=pltpu.CompilerParams(collective_id=0),
    )(x, w_mat, scale)


# device time: 113120 ns/iter; 1.7035x vs baseline; 1.7035x over previous
import jax
import jax.numpy as jnp
from jax import lax
from jax.experimental import pallas as pl
from jax.experimental.pallas import tpu as pltpu

N_DEV = 8


def kernel(x, w_mat, scale_x, scale_w):
    m, k = x.shape
    n = w_mat.shape[1]
    m_per = m // N_DEV
    nh = n // 2
    scale = (scale_x * scale_w).astype(jnp.float32).reshape(1, 1)

    def body(x_ref, w_ref, s_ref, out_ref,
             x_bf, w_bf,
             send_r, send_l, recv_r, recv_l,
             send_sems_r, send_sems_l, recv_sems_r, recv_sems_l):
        my = lax.axis_index("i")
        left = lax.rem(my + N_DEV - 1, N_DEV)
        right = lax.rem(my + 1, N_DEV)

        barrier_sem = pltpu.get_barrier_semaphore()
        for nbr in (left, right):
            pl.semaphore_signal(barrier_sem, inc=1, device_id=(nbr,),
                                device_id_type=pl.DeviceIdType.MESH)
        pl.semaphore_wait(barrier_sem, 2)

        x_bf[...] = x_ref[...].astype(jnp.bfloat16)
        w_bf[...] = w_ref[...].astype(jnp.bfloat16)

        def partial_r(c):
            return jnp.dot(x_bf[pl.ds(c * m_per, m_per), :], w_bf[:, :nh],
                           preferred_element_type=jnp.float32)

        def partial_l(c):
            return jnp.dot(x_bf[pl.ds(c * m_per, m_per), :], w_bf[:, nh:],
                           preferred_element_type=jnp.float32)

        send_r[0, :, :] = partial_r(lax.rem(my + N_DEV - 1, N_DEV)).astype(jnp.bfloat16)
        send_l[0, :, :] = partial_l(lax.rem(my + 1, N_DEV)).astype(jnp.bfloat16)

        descs_r = []
        descs_l = []
        for s in range(N_DEV - 1):
            c_r = lax.rem(my + (N_DEV - 2 - s), N_DEV)
            c_l = lax.rem(my + 2 + s, N_DEV)
            rdma_r = pltpu.make_async_remote_copy(
                src_ref=send_r.at[s % 2], dst_ref=recv_r.at[s],
                send_sem=send_sems_r.at[s], recv_sem=recv_sems_r.at[s],
                device_id=(right,), device_id_type=pl.DeviceIdType.MESH,
            )
            rdma_l = pltpu.make_async_remote_copy(
                src_ref=send_l.at[s % 2], dst_ref=recv_l.at[s],
                send_sem=send_sems_l.at[s], recv_sem=recv_sems_l.at[s],
                device_id=(left,), device_id_type=pl.DeviceIdType.MESH,
            )
            rdma_r.start()
            rdma_l.start()
            descs_r.append(rdma_r)
            descs_l.append(rdma_l)

            pr = partial_r(c_r)
            pl_ = partial_l(c_l)

            rdma_r.wait_recv()
            rdma_l.wait_recv()
            if s < N_DEV - 2:
                if s >= 1:
                    descs_r[s - 1].wait_send()
                    descs_l[s - 1].wait_send()
                send_r[(s + 1) % 2, :, :] = (
                    recv_r[s, :, :].astype(jnp.float32) + pr
                ).astype(jnp.bfloat16)
                send_l[(s + 1) % 2, :, :] = (
                    recv_l[s, :, :].astype(jnp.float32) + pl_
                ).astype(jnp.bfloat16)
            else:
                yr = (recv_r[s, :, :].astype(jnp.float32) + pr) * s_ref[0, 0]
                yl = (recv_l[s, :, :].astype(jnp.float32) + pl_) * s_ref[0, 0]
                out_ref[:, :nh] = yr * jax.nn.sigmoid(yr)
                out_ref[:, nh:] = yl * jax.nn.sigmoid(yl)

        for s in (N_DEV - 3, N_DEV - 2):
            descs_r[s].wait_send()
            descs_l[s].wait_send()

    return pl.pallas_call(
        body,
        out_shape=jax.ShapeDtypeStruct((m_per, n), jnp.float32),
        in_specs=[
            pl.BlockSpec(memory_space=pltpu.VMEM),
            pl.BlockSpec(memory_space=pltpu.VMEM),
            pl.BlockSpec(memory_space=pltpu.SMEM),
        ],
        out_specs=pl.BlockSpec(memory_space=pltpu.VMEM),
        scratch_shapes=[
            pltpu.VMEM((m, x.shape[1]), jnp.bfloat16),
            pltpu.VMEM((x.shape[1], n), jnp.bfloat16),
            pltpu.VMEM((2, m_per, n // 2), jnp.bfloat16),
            pltpu.VMEM((2, m_per, n // 2), jnp.bfloat16),
            pltpu.VMEM((N_DEV - 1, m_per, n // 2), jnp.bfloat16),
            pltpu.VMEM((N_DEV - 1, m_per, n // 2), jnp.bfloat16),
            pltpu.SemaphoreType.DMA((N_DEV - 1,)),
            pltpu.SemaphoreType.DMA((N_DEV - 1,)),
            pltpu.SemaphoreType.DMA((N_DEV - 1,)),
            pltpu.SemaphoreType.DMA((N_DEV - 1,)),
        ],
        compiler_params=pltpu.CompilerParams(collective_id=0),
    )(x, w_mat, scale)


# device time: 94962 ns/iter; 2.0292x vs baseline; 1.1912x over previous
import jax
import jax.numpy as jnp
from jax import lax
from jax.experimental import pallas as pl
from jax.experimental.pallas import tpu as pltpu

N_DEV = 8
N_FLOW = 4


def kernel(x, w_mat, scale_x, scale_w):
    m, k = x.shape
    n = w_mat.shape[1]
    m_per = m // N_DEV
    nq = n // N_FLOW
    scale = (scale_x * scale_w).astype(jnp.float32).reshape(1, 1)

    def body(x_ref, w_ref, s_ref, out_ref, x_bf, w_bf,
             send_bufs, recv_bufs, send_sems, recv_sems):
        my = lax.axis_index("i")
        left = lax.rem(my + N_DEV - 1, N_DEV)
        right = lax.rem(my + 1, N_DEV)

        barrier_sem = pltpu.get_barrier_semaphore()
        for nbr in (left, right):
            pl.semaphore_signal(barrier_sem, inc=1, device_id=(nbr,),
                                device_id_type=pl.DeviceIdType.MESH)
        pl.semaphore_wait(barrier_sem, 2)

        x_bf[...] = x_ref[...].astype(jnp.bfloat16)
        w_bf[...] = w_ref[...].astype(jnp.bfloat16)

        def pchunk(c, f):
            return jnp.dot(x_bf[pl.ds(c * m_per, m_per), :],
                           w_bf[:, f * nq:(f + 1) * nq],
                           preferred_element_type=jnp.float32)

        def c_in(f, s):
            if f < 2:
                return lax.rem(my + (N_DEV - 2 - s), N_DEV)
            return lax.rem(my + 2 + s, N_DEV)

        def rd(f, s):
            return pltpu.make_async_remote_copy(
                src_ref=send_bufs.at[f, s % 2],
                dst_ref=recv_bufs.at[f, s],
                send_sem=send_sems.at[f, s],
                recv_sem=recv_sems.at[f, s],
                device_id=(right if f < 2 else left,),
                device_id_type=pl.DeviceIdType.MESH,
            )

        descs = {}

        for f in range(N_FLOW):
            c0 = lax.rem(my + (N_DEV - 1 if f < 2 else 1), N_DEV)
            send_bufs[f, 0] = pchunk(c0, f).astype(jnp.bfloat16)
            d = rd(f, 0)
            descs[(f, 0)] = d
            d.start()

        for s in range(N_DEV - 1):
            for f in (0, 2, 1, 3):
                p = pchunk(c_in(f, s), f)
                d = descs[(f, s)]
                d.wait_recv()
                if s < N_DEV - 2:
                    if s >= 1:
                        descs[(f, s - 1)].wait_send()
                    send_bufs[f, (s + 1) % 2] = recv_bufs[f, s] + p.astype(jnp.bfloat16)
                    nd = rd(f, s + 1)
                    descs[(f, s + 1)] = nd
                    nd.start()
                else:
                    y = (recv_bufs[f, s].astype(jnp.float32) + p) * s_ref[0, 0]
                    out_ref[:, f * nq:(f + 1) * nq] = y * jax.nn.sigmoid(y)

        for f in range(N_FLOW):
            descs[(f, N_DEV - 3)].wait_send()
            descs[(f, N_DEV - 2)].wait_send()

    return pl.pallas_call(
        body,
        out_shape=jax.ShapeDtypeStruct((m_per, n), jnp.float32),
        in_specs=[
            pl.BlockSpec(memory_space=pltpu.VMEM),
            pl.BlockSpec(memory_space=pltpu.VMEM),
            pl.BlockSpec(memory_space=pltpu.SMEM),
        ],
        out_specs=pl.BlockSpec(memory_space=pltpu.VMEM),
        scratch_shapes=[
            pltpu.VMEM((m, k), jnp.bfloat16),
            pltpu.VMEM((k, n), jnp.bfloat16),
            pltpu.VMEM((N_FLOW, 2, m_per, nq), jnp.bfloat16),
            pltpu.VMEM((N_FLOW, N_DEV - 1, m_per, nq), jnp.bfloat16),
            pltpu.SemaphoreType.DMA((N_FLOW, N_DEV - 1)),
            pltpu.SemaphoreType.DMA((N_FLOW, N_DEV - 1)),
        ],
        compiler_params=pltpu.CompilerParams(collective_id=0),
    )(x, w_mat, scale)
